# baseline (speedup 1.0000x reference)
import functools

import jax
import jax.numpy as jnp
from jax import lax
from jax.experimental import pallas as pl
from jax.experimental.pallas import tpu as pltpu
from jax.experimental.pallas import tpu_sc as plsc

EXTENT = 0.3
TBL = 144
CHUNK = 128
NC, NS = 2, 16
NW = NC * NS
MB = 400
GRP = 8
KP = 16


def _sc_gather(table, idx):
    B = idx.shape[0]
    per_w = B // NW
    n_chunks = per_w // CHUNK
    mesh = plsc.VectorSubcoreMesh(core_axis_name="c", subcore_axis_name="s")

    @functools.partial(
        pl.kernel,
        mesh=mesh,
        out_type=jax.ShapeDtypeStruct((B, TBL), jnp.float32),
        compiler_params=pltpu.CompilerParams(use_tc_tiling_on_sc=False),
        scratch_types=[
            pltpu.VMEM((CHUNK,), jnp.int32),
            pltpu.VMEM((CHUNK, TBL), jnp.float32),
            pltpu.SemaphoreType.DMA,
        ],
    )
    def gather_kernel(table_hbm, idx_hbm, out_hbm, idx_v, rows_v, sem):
        wid = lax.axis_index("s") * NC + lax.axis_index("c")
        base = wid * per_w

        @pl.loop(0, n_chunks)
        def _(c):
            off = base + c * CHUNK
            pltpu.sync_copy(idx_hbm.at[pl.ds(off, CHUNK)], idx_v)
            pltpu.async_copy(table_hbm.at[idx_v], rows_v, sem).wait()
            pltpu.sync_copy(rows_v, out_hbm.at[pl.ds(off, CHUNK)])

    return gather_kernel(table, idx)


def _make_tc_body(mb, d, k):
    rows_g = GRP * d
    cols_g = GRP * KP
    n_grp = mb // GRP

    def tc_body(gath_ref, outp_ref, kpt_ref, kvf_ref, out_ref, w_ref, wf_ref):
        pts = gath_ref[:, 128:131]
        op = outp_ref[...]
        opr = jnp.broadcast_to(op[:, None, :], (mb, d, 3)).reshape(mb * d, 3)
        sq = jnp.zeros((mb * d, KP), jnp.float32)
        for c in range(3):
            dc = pts[:, c:c + 1] - opr[:, c:c + 1]
            sq = sq + (dc - kpt_ref[c:c + 1, :]) ** 2
        w_ref[...] = jnp.maximum(1.0 - jnp.sqrt(sq) / EXTENT, 0.0)

        rgrp = lax.broadcasted_iota(jnp.int32, (rows_g, cols_g), 0) // d
        cgrp = lax.broadcasted_iota(jnp.int32, (rows_g, cols_g), 1) // KP
        mask = (rgrp == cgrp).astype(jnp.float32)

        def body(g, _):
            wg = w_ref[pl.ds(g * rows_g, rows_g), :]
            tg = jnp.broadcast_to(
                wg[:, None, :], (rows_g, GRP, KP)).reshape(rows_g, cols_g)
            ag = tg * mask
            fg = gath_ref[pl.ds(g * rows_g, rows_g), 0:128]
            wf2 = lax.dot_general(ag, fg, (((0,), (0,)), ((), ())),
                                  preferred_element_type=jnp.float32)
            wf_ref[pl.ds(g * GRP, GRP), :, :] = wf2.reshape(GRP, KP, 128)
            return 0

        lax.fori_loop(0, n_grp, body, 0)

        acc = jnp.zeros((mb, 128), jnp.float32)
        for kk in range(k):
            acc = acc + jnp.dot(wf_ref[:, kk, :], kvf_ref[kk],
                                preferred_element_type=jnp.float32)
        out_ref[...] = acc
    return tc_body


def kernel(points, features, output_points, neighbor_indices, k_points, k_values):
    n, f = features.shape
    m, d = neighbor_indices.shape
    k = k_values.shape[0]
    c_out = k_values.shape[2]

    table = jnp.concatenate(
        [features, points,
         jnp.zeros((n, TBL - f - 3), jnp.float32)], axis=1)
    b = m * d
    grain = NW * CHUNK
    b_pad = ((b + grain - 1) // grain) * grain
    idx = jnp.pad(neighbor_indices.reshape(-1).astype(jnp.int32),
                  (0, b_pad - b))

    kpt = jnp.full((4, KP), 1e6, jnp.float32)
    kpt = kpt.at[0:3, 0:k].set(k_points.T)

    gathered = _sc_gather(table, idx)

    out = pl.pallas_call(
        _make_tc_body(MB, d, k),
        grid=(m // MB,),
        in_specs=[
            pl.BlockSpec((MB * d, TBL), lambda i: (i, 0)),
            pl.BlockSpec((MB, 3), lambda i: (i, 0)),
            pl.BlockSpec((4, KP), lambda i: (0, 0)),
            pl.BlockSpec((k, f, c_out), lambda i: (0, 0, 0)),
        ],
        out_specs=pl.BlockSpec((MB, c_out), lambda i: (i, 0)),
        out_shape=jax.ShapeDtypeStruct((m, c_out), jnp.float32),
        scratch_shapes=[
            pltpu.VMEM((MB * d, KP), jnp.float32),
            pltpu.VMEM((MB, KP, 128), jnp.float32),
        ],
    )(gathered, output_points, kpt, k_values)
    return out


# speedup vs baseline: 1.0211x; 1.0211x over previous
import functools

import jax
import jax.numpy as jnp
from jax import lax
from jax.experimental import pallas as pl
from jax.experimental.pallas import tpu as pltpu
from jax.experimental.pallas import tpu_sc as plsc

EXTENT = 0.3
TBL = 144
CHUNK = 128
NC, NS = 2, 16
NW = NC * NS
MB = 400


def _sc_gather(table, idx):
    B = idx.shape[0]
    per_w = B // NW
    n_chunks = per_w // CHUNK
    assert n_chunks % 2 == 0
    mesh = plsc.VectorSubcoreMesh(core_axis_name="c", subcore_axis_name="s")

    @functools.partial(
        pl.kernel,
        mesh=mesh,
        out_type=jax.ShapeDtypeStruct((B, TBL), jnp.float32),
        compiler_params=pltpu.CompilerParams(use_tc_tiling_on_sc=False),
        scratch_types=[
            pltpu.VMEM((2, CHUNK), jnp.int32),
            pltpu.VMEM((2, CHUNK, TBL), jnp.float32),
            pltpu.SemaphoreType.DMA,
            pltpu.SemaphoreType.DMA,
            pltpu.SemaphoreType.DMA,
            pltpu.SemaphoreType.DMA,
            pltpu.SemaphoreType.DMA,
        ],
    )
    def gather_kernel(table_hbm, idx_hbm, out_hbm, idx_v, rows_v,
                      sem_g, sem_i0, sem_i1, sem_o0, sem_o1):
        wid = lax.axis_index("s") * NC + lax.axis_index("c")
        base = wid * per_w
        sem_i = (sem_i0, sem_i1)
        sem_o = (sem_o0, sem_o1)

        pltpu.async_copy(idx_hbm.at[pl.ds(base, CHUNK)], idx_v.at[0],
                         sem_i[0])

        @pl.loop(0, n_chunks, step=2)
        def _(c0):
            for b in range(2):
                c = c0 + b
                off = c * CHUNK
                nb = 1 - b
                @pl.when(c + 1 < n_chunks)
                def _():
                    pltpu.async_copy(
                        idx_hbm.at[pl.ds(base + off + CHUNK, CHUNK)],
                        idx_v.at[nb], sem_i[nb])
                pltpu.make_async_copy(
                    idx_hbm.at[pl.ds(base + off, CHUNK)], idx_v.at[b],
                    sem_i[b]).wait()
                @pl.when(c >= 2)
                def _():
                    pltpu.make_async_copy(
                        rows_v.at[b],
                        out_hbm.at[pl.ds(base + off - 2 * CHUNK, CHUNK)],
                        sem_o[b]).wait()
                pltpu.async_copy(
                    table_hbm.at[idx_v.at[b]], rows_v.at[b], sem_g).wait()
                pltpu.async_copy(
                    rows_v.at[b], out_hbm.at[pl.ds(base + off, CHUNK)],
                    sem_o[b])

        for b in range(2):
            c = n_chunks - 2 + b
            pltpu.make_async_copy(
                rows_v.at[b], out_hbm.at[pl.ds(base + c * CHUNK, CHUNK)],
                sem_o[b]).wait()

    return gather_kernel(table, idx)


def _make_tc_body(mb, d, k):
    def tc_body(gath_ref, outp_ref, kpt_ref, kv_ref, out_ref):
        feats = gath_ref[:, 0:128]
        pts = gath_ref[:, 128:131]
        op = outp_ref[...]
        opr = jnp.broadcast_to(op[:, None, :], (mb, d, 3)).reshape(mb * d, 3)
        sq = jnp.zeros((mb * d, 16), jnp.float32)
        for c in range(3):
            dc = pts[:, c:c + 1] - opr[:, c:c + 1]
            sq = sq + (dc - kpt_ref[c:c + 1, :]) ** 2
        w = jnp.maximum(1.0 - jnp.sqrt(sq) / EXTENT, 0.0)
        acc = jnp.zeros((mb, 128), jnp.float32)
        for j in range(k):
            p = w[:, j:j + 1] * feats
            wfj = p.reshape(mb, d, 128).sum(axis=1)
            acc = acc + jnp.dot(wfj, kv_ref[j],
                                preferred_element_type=jnp.float32)
        out_ref[...] = acc
    return tc_body


def kernel(points, features, output_points, neighbor_indices, k_points, k_values):
    n, f = features.shape
    m, d = neighbor_indices.shape
    k = k_values.shape[0]
    c_out = k_values.shape[2]

    table = jnp.concatenate(
        [features, points,
         jnp.zeros((n, TBL - f - 3), jnp.float32)], axis=1)
    b = m * d
    grain = NW * CHUNK * 2
    b_pad = ((b + grain - 1) // grain) * grain
    idx = jnp.pad(neighbor_indices.reshape(-1).astype(jnp.int32),
                  (0, b_pad - b))

    kpt = jnp.full((4, 16), 1e6, jnp.float32)
    kpt = kpt.at[0:3, 0:k].set(k_points.T)

    gathered = _sc_gather(table, idx)

    out = pl.pallas_call(
        _make_tc_body(MB, d, k),
        grid=(m // MB,),
        in_specs=[
            pl.BlockSpec((MB * d, TBL), lambda i: (i, 0)),
            pl.BlockSpec((MB, 3), lambda i: (i, 0)),
            pl.BlockSpec((4, 16), lambda i: (0, 0)),
            pl.BlockSpec((k, f, c_out), lambda i: (0, 0, 0)),
        ],
        out_specs=pl.BlockSpec((MB, c_out), lambda i: (i, 0)),
        out_shape=jax.ShapeDtypeStruct((m, c_out), jnp.float32),
    )(gathered, output_points, kpt, k_values)
    return out


# speedup vs baseline: 1.1244x; 1.1012x over previous
import functools

import jax
import jax.numpy as jnp
from jax import lax
from jax.experimental import pallas as pl
from jax.experimental.pallas import tpu as pltpu
from jax.experimental.pallas import tpu_sc as plsc

EXTENT = 0.3
PTC = 16
CHUNK = 128
NC, NS = 2, 16
NW = NC * NS
MB = 400


def _sc_gather(featb, coords, idx):
    B = idx.shape[0]
    per_w = B // NW
    n_chunks = per_w // CHUNK
    mesh = plsc.VectorSubcoreMesh(core_axis_name="c", subcore_axis_name="s")

    @functools.partial(
        pl.kernel,
        mesh=mesh,
        out_type=(
            jax.ShapeDtypeStruct((B, featb.shape[1]), jnp.bfloat16),
            jax.ShapeDtypeStruct((B, PTC), jnp.float32),
        ),
        compiler_params=pltpu.CompilerParams(use_tc_tiling_on_sc=False),
        scratch_types=[
            pltpu.VMEM((CHUNK,), jnp.int32),
            pltpu.VMEM((CHUNK, featb.shape[1]), jnp.bfloat16),
            pltpu.VMEM((CHUNK, PTC), jnp.float32),
            pltpu.SemaphoreType.DMA,
            pltpu.SemaphoreType.DMA,
        ],
    )
    def gather_kernel(featb_hbm, coords_hbm, idx_hbm, outf_hbm, outp_hbm,
                      idx_v, rowsf_v, rowsp_v, sem_f, sem_p):
        wid = lax.axis_index("s") * NC + lax.axis_index("c")
        base = wid * per_w

        @pl.loop(0, n_chunks)
        def _(c):
            off = base + c * CHUNK
            pltpu.sync_copy(idx_hbm.at[pl.ds(off, CHUNK)], idx_v)
            cf = pltpu.async_copy(featb_hbm.at[idx_v], rowsf_v, sem_f)
            cp = pltpu.async_copy(coords_hbm.at[idx_v], rowsp_v, sem_p)
            cf.wait()
            cp.wait()
            pltpu.sync_copy(rowsf_v, outf_hbm.at[pl.ds(off, CHUNK)])
            pltpu.sync_copy(rowsp_v, outp_hbm.at[pl.ds(off, CHUNK)])

    return gather_kernel(featb, coords, idx)


def _make_tc_body(mb, d, k):
    def tc_body(gf_ref, gp_ref, outp_ref, kpt_ref, kv_ref, out_ref):
        feats = gf_ref[...].astype(jnp.float32)
        pts = gp_ref[:, 0:3]
        op = outp_ref[...]
        opr = jnp.broadcast_to(op[:, None, :], (mb, d, 3)).reshape(mb * d, 3)
        sq = jnp.zeros((mb * d, 16), jnp.float32)
        for c in range(3):
            dc = pts[:, c:c + 1] - opr[:, c:c + 1]
            sq = sq + (dc - kpt_ref[c:c + 1, :]) ** 2
        w = jnp.maximum(1.0 - jnp.sqrt(sq) / EXTENT, 0.0)
        acc = jnp.zeros((mb, 128), jnp.float32)
        for j in range(k):
            p = w[:, j:j + 1] * feats
            wfj = p.reshape(mb, d, 128).sum(axis=1)
            acc = acc + jnp.dot(wfj, kv_ref[j],
                                preferred_element_type=jnp.float32)
        out_ref[...] = acc
    return tc_body


def kernel(points, features, output_points, neighbor_indices, k_points, k_values):
    n, f = features.shape
    m, d = neighbor_indices.shape
    k = k_values.shape[0]
    c_out = k_values.shape[2]

    featb = features.astype(jnp.bfloat16)
    coords = jnp.concatenate(
        [points, jnp.zeros((n, PTC - 3), jnp.float32)], axis=1)
    b = m * d
    grain = NW * CHUNK
    b_pad = ((b + grain - 1) // grain) * grain
    idx = jnp.pad(neighbor_indices.reshape(-1).astype(jnp.int32),
                  (0, b_pad - b))

    kpt = jnp.full((4, 16), 1e6, jnp.float32)
    kpt = kpt.at[0:3, 0:k].set(k_points.T)

    gf, gp = _sc_gather(featb, coords, idx)

    out = pl.pallas_call(
        _make_tc_body(MB, d, k),
        grid=(m // MB,),
        in_specs=[
            pl.BlockSpec((MB * d, f), lambda i: (i, 0)),
            pl.BlockSpec((MB * d, PTC), lambda i: (i, 0)),
            pl.BlockSpec((MB, 3), lambda i: (i, 0)),
            pl.BlockSpec((4, 16), lambda i: (0, 0)),
            pl.BlockSpec((k, f, c_out), lambda i: (0, 0, 0)),
        ],
        out_specs=pl.BlockSpec((MB, c_out), lambda i: (i, 0)),
        out_shape=jax.ShapeDtypeStruct((m, c_out), jnp.float32),
    )(gf, gp, output_points, kpt, k_values)
    return out


# speedup vs baseline: 1.3892x; 1.2355x over previous
import functools

import jax
import jax.numpy as jnp
from jax import lax
from jax.experimental import pallas as pl
from jax.experimental.pallas import tpu as pltpu
from jax.experimental.pallas import tpu_sc as plsc

EXTENT = 0.3
PTC = 16
CHUNK = 128
NC, NS = 2, 16
NW = NC * NS
MB = 400


def _sc_gather(featb, coords, idx):
    B = idx.shape[0]
    per_w = B // NW
    n_chunks = per_w // CHUNK
    mesh = plsc.VectorSubcoreMesh(core_axis_name="c", subcore_axis_name="s")

    @functools.partial(
        pl.kernel,
        mesh=mesh,
        out_type=(
            jax.ShapeDtypeStruct((B, featb.shape[1]), jnp.bfloat16),
            jax.ShapeDtypeStruct((B, PTC), jnp.float32),
        ),
        compiler_params=pltpu.CompilerParams(use_tc_tiling_on_sc=False),
        scratch_types=[
            pltpu.VMEM((CHUNK,), jnp.int32),
            pltpu.VMEM((CHUNK, featb.shape[1]), jnp.bfloat16),
            pltpu.VMEM((CHUNK, PTC), jnp.float32),
            pltpu.SemaphoreType.DMA,
            pltpu.SemaphoreType.DMA,
        ],
    )
    def gather_kernel(featb_hbm, coords_hbm, idx_hbm, outf_hbm, outp_hbm,
                      idx_v, rowsf_v, rowsp_v, sem_f, sem_p):
        wid = lax.axis_index("s") * NC + lax.axis_index("c")
        base = wid * per_w

        @pl.loop(0, n_chunks)
        def _(c):
            off = base + c * CHUNK
            pltpu.sync_copy(idx_hbm.at[pl.ds(off, CHUNK)], idx_v)
            cf = pltpu.async_copy(featb_hbm.at[idx_v], rowsf_v, sem_f)
            cp = pltpu.async_copy(coords_hbm.at[idx_v], rowsp_v, sem_p)
            cf.wait()
            cp.wait()
            pltpu.sync_copy(rowsf_v, outf_hbm.at[pl.ds(off, CHUNK)])
            pltpu.sync_copy(rowsp_v, outp_hbm.at[pl.ds(off, CHUNK)])

    return gather_kernel(featb, coords, idx)


def _make_tc_body(mb, d, k):
    def tc_body(gf_ref, gp_ref, outp_ref, kpt_ref, kv_ref, out_ref):
        featsb = gf_ref[...]
        pts = gp_ref[:, 0:3]
        op = outp_ref[...]
        opr = jnp.broadcast_to(op[:, None, :], (mb, d, 3)).reshape(mb * d, 3)
        sq = jnp.zeros((mb * d, 16), jnp.float32)
        for c in range(3):
            dc = pts[:, c:c + 1] - opr[:, c:c + 1]
            sq = sq + (dc - kpt_ref[c:c + 1, :]) ** 2
        w = jnp.maximum(1.0 - jnp.sqrt(sq) / EXTENT, 0.0)
        wb = w.astype(jnp.bfloat16)
        acc = jnp.zeros((mb, 128), jnp.float32)
        for j in range(k):
            p = wb[:, j:j + 1] * featsb
            wfj = p.reshape(mb, d, 128).sum(axis=1)
            acc = acc + jnp.dot(wfj.astype(jnp.float32), kv_ref[j],
                                preferred_element_type=jnp.float32)
        out_ref[...] = acc
    return tc_body


def kernel(points, features, output_points, neighbor_indices, k_points, k_values):
    n, f = features.shape
    m, d = neighbor_indices.shape
    k = k_values.shape[0]
    c_out = k_values.shape[2]

    featb = features.astype(jnp.bfloat16)
    coords = jnp.concatenate(
        [points, jnp.zeros((n, PTC - 3), jnp.float32)], axis=1)
    b = m * d
    grain = NW * CHUNK
    b_pad = ((b + grain - 1) // grain) * grain
    idx = jnp.pad(neighbor_indices.reshape(-1).astype(jnp.int32),
                  (0, b_pad - b))

    kpt = jnp.full((4, 16), 1e6, jnp.float32)
    kpt = kpt.at[0:3, 0:k].set(k_points.T)

    gf, gp = _sc_gather(featb, coords, idx)

    out = pl.pallas_call(
        _make_tc_body(MB, d, k),
        grid=(m // MB,),
        in_specs=[
            pl.BlockSpec((MB * d, f), lambda i: (i, 0)),
            pl.BlockSpec((MB * d, PTC), lambda i: (i, 0)),
            pl.BlockSpec((MB, 3), lambda i: (i, 0)),
            pl.BlockSpec((4, 16), lambda i: (0, 0)),
            pl.BlockSpec((k, f, c_out), lambda i: (0, 0, 0)),
        ],
        out_specs=pl.BlockSpec((MB, c_out), lambda i: (i, 0)),
        out_shape=jax.ShapeDtypeStruct((m, c_out), jnp.float32),
    )(gf, gp, output_points, kpt, k_values)
    return out
